# Initial kernel scaffold; baseline (speedup 1.0000x reference)
import jax
import jax.numpy as jnp
from jax.experimental import pallas as pl


def kernel(x, edge_index, W1l, b1l, W1r, W2l, b2l, W2r):
    raise NotImplementedError("write your pallas kernel here")


# speedup vs baseline: 5.1155x; 5.1155x over previous
import jax
import jax.numpy as jnp
from jax import lax
from jax.experimental import pallas as pl
from jax.experimental.pallas import tpu as pltpu
from jax.experimental.pallas import tpu_sc as plsc

N = 10000
D = 256
H = 256
E = 160000
HALF = 128

NS = 16
NC = 2
BB = 128
NB = 79
EPT = NB * BB
E_PAD = EPT * NS
N_ACC = 10240
RPT = N_ACC // NS
RB = 1024
GRID = N_ACC // RB


def _dot(a, b):
    return lax.dot_general(a, b, (((1,), (1,)), ((), ())),
                           preferred_element_type=jnp.float32)


def _mm1_body(x_ref, wl_ref, wr_ref, b_ref, t_ref, xr_ref):
    xb = x_ref[...]
    t_ref[...] = _dot(xb, wl_ref[...])
    xr_ref[...] = _dot(xb, wr_ref[...]) + b_ref[...]


def _mm1(x, W1l, W1r, b1l):
    return pl.pallas_call(
        _mm1_body,
        grid=(NC, GRID),
        in_specs=[
            pl.BlockSpec((RB, D), lambda k, i: (i, 0)),
            pl.BlockSpec((HALF, D), lambda k, i: (k, 0)),
            pl.BlockSpec((HALF, D), lambda k, i: (k, 0)),
            pl.BlockSpec((1, HALF), lambda k, i: (0, k)),
        ],
        out_specs=[
            pl.BlockSpec((RB, HALF), lambda k, i: (k * GRID + i, 0)),
            pl.BlockSpec((RB, HALF), lambda k, i: (i, k)),
        ],
        out_shape=[
            jax.ShapeDtypeStruct((NC * N_ACC, HALF), jnp.float32),
            jax.ShapeDtypeStruct((N_ACC, H), jnp.float32),
        ],
    )(x, W1l, W1r, b1l)


NW = NC * NS
NBC = (NB + 1) // 2


def _cnt_body(dsts, cntp, idv, rows, acc, sem):
    del sem
    c = lax.axis_index("c")
    s = lax.axis_index("s")

    def _fill(val):
        def _f(k, carry):
            i = k // (HALF // 16)
            m = (k % (HALF // 16)) * 16
            rows[i, pl.ds(m, 16)] = val
            return carry
        lax.fori_loop(0, BB * HALF // 16, _f, 0)

    _fill(jnp.zeros((16,), jnp.float32))

    def _zacc(bb, carry):
        pltpu.sync_copy(rows, acc.at[pl.ds(s * RPT + bb * BB, BB)])
        return carry
    lax.fori_loop(0, RPT // BB, _zacc, 0)

    _fill(jnp.ones((16,), jnp.float32))

    pltpu.sync_copy(dsts.at[s], idv)
    plsc.subcore_barrier()

    def _step(k, carry):
        j2 = 2 * k + c

        @pl.when(j2 < NB)
        def _():
            pltpu.sync_copy(rows, acc.at[idv.at[j2]], add=True)
        return carry
    lax.fori_loop(0, NBC, _step, 0)

    plsc.subcore_barrier()
    r0 = s * RPT
    pltpu.sync_copy(acc.at[pl.ds(r0, RPT)],
                    cntp.at[pl.ds(c * N_ACC + r0, RPT)])


_sc_cnt = pl.kernel(
    _cnt_body,
    jax.ShapeDtypeStruct((NC * N_ACC, HALF), jnp.float32),
    mesh=plsc.VectorSubcoreMesh(core_axis_name="c", subcore_axis_name="s"),
    scratch_types=[
        pltpu.VMEM((NB, BB), jnp.int32),
        pltpu.VMEM((BB, HALF), jnp.float32),
        pltpu.VMEM_SHARED((N_ACC, HALF), jnp.float32),
        pltpu.SemaphoreType.DMA,
    ],
)


def _sc_body(tcat, srcs2, dsts, aggcat, isv, idv, rows, acc, sem):
    c = lax.axis_index("c")
    s = lax.axis_index("s")

    def _zrow(k, carry):
        i = k // (HALF // 16)
        m = (k % (HALF // 16)) * 16
        rows[i, pl.ds(m, 16)] = jnp.zeros((16,), jnp.float32)
        return carry
    lax.fori_loop(0, BB * HALF // 16, _zrow, 0)

    def _zacc(b, carry):
        pltpu.sync_copy(rows, acc.at[pl.ds(s * RPT + b * BB, BB)])
        return carry
    lax.fori_loop(0, RPT // BB, _zacc, 0)

    pltpu.sync_copy(srcs2.at[c].at[s], isv)
    pltpu.sync_copy(dsts.at[s], idv)

    plsc.subcore_barrier()

    def _step(j, carry):
        pltpu.async_copy(tcat.at[isv.at[j]], rows, sem).wait()
        pltpu.sync_copy(rows, acc.at[idv.at[j]], add=True)
        return carry
    lax.fori_loop(0, NB, _step, 0)

    plsc.subcore_barrier()

    r0 = s * RPT
    pltpu.sync_copy(acc.at[pl.ds(r0, RPT)],
                    aggcat.at[pl.ds(c * N_ACC + r0, RPT)])


_sc_agg = pl.kernel(
    _sc_body,
    jax.ShapeDtypeStruct((NC * N_ACC, HALF), jnp.float32),
    mesh=plsc.VectorSubcoreMesh(core_axis_name="c", subcore_axis_name="s"),
    scratch_types=[
        pltpu.VMEM((NB, BB), jnp.int32),
        pltpu.VMEM((NB, BB), jnp.int32),
        pltpu.VMEM((BB, HALF), jnp.float32),
        pltpu.VMEM_SHARED((N_ACC, HALF), jnp.float32),
        pltpu.SemaphoreType.DMA,
    ],
)


def _mm2_body(aa_ref, ab_ref, c0_ref, c1_ref, xr_ref, wl_ref, wr_ref, b_ref,
              t_ref, hr_ref):
    inv = 1.0 / jnp.maximum(c0_ref[:, 0:1] + c1_ref[:, 0:1], 1.0)
    h = jnp.concatenate([aa_ref[...], ab_ref[...]], axis=1) * inv + xr_ref[...]
    h = jnp.maximum(h, 0.0)
    t_ref[...] = _dot(h, wl_ref[...])
    hr_ref[...] = _dot(h, wr_ref[...]) + b_ref[...]


def _mm2(aggcat, cnt, xr, W2l, W2r, b2l):
    return pl.pallas_call(
        _mm2_body,
        grid=(NC, GRID),
        in_specs=[
            pl.BlockSpec((RB, HALF), lambda k, i: (i, 0)),
            pl.BlockSpec((RB, HALF), lambda k, i: (GRID + i, 0)),
            pl.BlockSpec((RB, HALF), lambda k, i: (i, 0)),
            pl.BlockSpec((RB, HALF), lambda k, i: (N_ACC // RB + i, 0)),
            pl.BlockSpec((RB, H), lambda k, i: (i, 0)),
            pl.BlockSpec((HALF, H), lambda k, i: (k, 0)),
            pl.BlockSpec((HALF, H), lambda k, i: (k, 0)),
            pl.BlockSpec((1, HALF), lambda k, i: (0, k)),
        ],
        out_specs=[
            pl.BlockSpec((RB, HALF), lambda k, i: (k * GRID + i, 0)),
            pl.BlockSpec((RB, HALF), lambda k, i: (i, k)),
        ],
        out_shape=[
            jax.ShapeDtypeStruct((NC * N_ACC, HALF), jnp.float32),
            jax.ShapeDtypeStruct((N_ACC, H), jnp.float32),
        ],
    )(aggcat, aggcat, cnt, cnt, xr, W2l, W2r, b2l)


def _mm3_body(aa_ref, ab_ref, c0_ref, c1_ref, hr_ref, out_ref):
    inv = 1.0 / jnp.maximum(c0_ref[:, 0:1] + c1_ref[:, 0:1], 1.0)
    out_ref[...] = (jnp.concatenate([aa_ref[...], ab_ref[...]], axis=1) * inv
                    + hr_ref[...])


def _mm3(aggcat, cnt, hr):
    return pl.pallas_call(
        _mm3_body,
        grid=(GRID,),
        in_specs=[
            pl.BlockSpec((RB, HALF), lambda i: (i, 0)),
            pl.BlockSpec((RB, HALF), lambda i: (GRID + i, 0)),
            pl.BlockSpec((RB, HALF), lambda i: (i, 0)),
            pl.BlockSpec((RB, HALF), lambda i: (N_ACC // RB + i, 0)),
            pl.BlockSpec((RB, H), lambda i: (i, 0)),
        ],
        out_specs=pl.BlockSpec((RB, H), lambda i: (i, 0)),
        out_shape=jax.ShapeDtypeStruct((N, H), jnp.float32),
    )(aggcat, aggcat, cnt, cnt, hr)


def kernel(x, edge_index, W1l, b1l, W1r, W2l, b2l, W2r):
    src = edge_index[0].astype(jnp.int32)
    dst = edge_index[1].astype(jnp.int32)
    ar = jnp.arange(E_PAD - E, dtype=jnp.int32)
    src_p = jnp.concatenate([src, ar % N])
    dst_p = jnp.concatenate([dst, N + 112 + (ar % 128)])
    srcs = src_p.reshape(NS, NB, BB)
    srcs2 = jnp.stack([srcs, srcs + N_ACC])
    dsts = dst_p.reshape(NS, NB, BB)

    b1 = b1l.reshape(1, H)
    b2 = b2l.reshape(1, H)

    t1, xr1 = _mm1(x, W1l, W1r, b1)
    cnt = _sc_cnt(dsts)
    agg1 = _sc_agg(t1, srcs2, dsts)
    t2, hr2 = _mm2(agg1, cnt, xr1, W2l, W2r, b2)
    agg2 = _sc_agg(t2, srcs2, dsts)
    return _mm3(agg2, cnt, hr2)
